# Initial kernel scaffold; baseline (speedup 1.0000x reference)
import jax
import jax.numpy as jnp
from jax.experimental import pallas as pl


def kernel(X, edge_index, W, b):
    raise NotImplementedError("write your pallas kernel here")


# speedup vs baseline: 23.8279x; 23.8279x over previous
import functools

import jax
import jax.numpy as jnp
from jax import lax
from jax.experimental import pallas as pl
from jax.experimental.pallas import tpu as pltpu
from jax.experimental.pallas import tpu_sc as plsc

N = 10000
E = 320000
D = 128

NC = 2
NS = 16
NW = NC * NS
EPW = E // NW
KC = 80
NCH = EPW // KC

RPT = 624
RLAST = N - RPT * (NS - 1)

_mesh = plsc.VectorSubcoreMesh(core_axis_name="c", subcore_axis_name="s")


@functools.partial(
    pl.kernel,
    out_type=jax.ShapeDtypeStruct((NC * N,), jnp.float32),
    mesh=_mesh,
    scratch_types=[
        pltpu.VMEM((NCH, KC), jnp.int32),
        pltpu.VMEM((KC,), jnp.float32),
        pltpu.VMEM((RLAST,), jnp.float32),
        pltpu.VMEM_SHARED((N,), jnp.float32),
    ],
)
def _deg_kernel(dst3d, out, idx_v, ones_v, stage_v, deg_sh):
    cid = lax.axis_index("c")
    sid = lax.axis_index("s")
    wid = cid * NS + sid
    for i in range(KC // 16):
        ones_v[pl.ds(i * 16, 16)] = jnp.ones((16,), jnp.float32)
    for i in range(RLAST // 16):
        stage_v[pl.ds(i * 16, 16)] = jnp.zeros((16,), jnp.float32)
    pltpu.sync_copy(stage_v.at[pl.ds(0, RPT)], deg_sh.at[pl.ds(sid * RPT, RPT)])

    @pl.when(sid == NS - 1)
    def _():
        pltpu.sync_copy(stage_v.at[pl.ds(RPT, RLAST - RPT)],
                        deg_sh.at[pl.ds(RPT * NS, RLAST - RPT)])

    pltpu.sync_copy(dst3d.at[wid], idx_v)
    plsc.subcore_barrier()

    def step(j, carry):
        pltpu.sync_copy(ones_v, deg_sh.at[idx_v.at[j]], add=True)
        return carry

    lax.fori_loop(0, NCH, step, 0)
    plsc.subcore_barrier()
    pltpu.sync_copy(deg_sh.at[pl.ds(sid * RPT, RPT)], stage_v.at[pl.ds(0, RPT)])
    pltpu.sync_copy(stage_v.at[pl.ds(0, RPT)],
                    out.at[pl.ds(cid * N + sid * RPT, RPT)])

    @pl.when(sid == NS - 1)
    def _():
        pltpu.sync_copy(deg_sh.at[pl.ds(RPT * NS, RLAST - RPT)],
                        stage_v.at[pl.ds(RPT, RLAST - RPT)])
        pltpu.sync_copy(stage_v.at[pl.ds(RPT, RLAST - RPT)],
                        out.at[pl.ds(cid * N + RPT * NS, RLAST - RPT)])


@functools.partial(
    pl.kernel,
    out_type=jax.ShapeDtypeStruct((NC, N, D), jnp.float32),
    mesh=_mesh,
    scratch_types=[
        pltpu.VMEM((EPW,), jnp.int32),
        pltpu.VMEM((NCH, KC), jnp.int32),
        pltpu.VMEM((KC, D), jnp.float32),
        pltpu.VMEM_SHARED((N, D), jnp.float32),
        pltpu.SemaphoreType.DMA,
    ],
)
def _gather_scatter_kernel(z, src, dst3d, zerosd, out,
                           src_v, dstidx_v, rows_v, acc_sh, gsem):
    cid = lax.axis_index("c")
    sid = lax.axis_index("s")
    wid = cid * NS + sid
    pltpu.sync_copy(zerosd.at[pl.ds(0, RPT)], acc_sh.at[pl.ds(sid * RPT, RPT)])

    @pl.when(sid == NS - 1)
    def _():
        pltpu.sync_copy(zerosd.at[pl.ds(RPT, RLAST - RPT)],
                        acc_sh.at[pl.ds(RPT * NS, RLAST - RPT)])

    pltpu.sync_copy(src.at[pl.ds(wid * EPW, EPW)], src_v)
    pltpu.sync_copy(dst3d.at[wid], dstidx_v)
    plsc.subcore_barrier()

    def step(j, carry):
        off = pl.multiple_of(j * KC, 8)
        pltpu.async_copy(z.at[src_v.at[pl.ds(off, KC)]], rows_v, gsem).wait()
        pltpu.sync_copy(rows_v, acc_sh.at[dstidx_v.at[j]], add=True)
        return carry

    lax.fori_loop(0, NCH, step, 0)
    plsc.subcore_barrier()
    pltpu.sync_copy(acc_sh.at[pl.ds(sid * RPT, RPT)],
                    out.at[cid, pl.ds(sid * RPT, RPT)])

    @pl.when(sid == NS - 1)
    def _():
        pltpu.sync_copy(acc_sh.at[pl.ds(RPT * NS, RLAST - RPT)],
                        out.at[cid, pl.ds(RPT * NS, RLAST - RPT)])


def _mm_body(x_ref, wt_ref, b_ref, deg_ref, z_ref):
    y = jnp.dot(x_ref[...], wt_ref[...], preferred_element_type=jnp.float32)
    y = y + b_ref[...]
    deg = deg_ref[...]
    d = deg[0] + deg[1] + 1.0
    z_ref[...] = y * lax.rsqrt(d)


def _final_body(acc_ref, z_ref, deg_ref, o_ref):
    deg = deg_ref[...]
    d = deg[0] + deg[1] + 1.0
    norm = lax.rsqrt(d)
    acc = acc_ref[...]
    s = (acc[0] + acc[1] + z_ref[...]) * norm
    o_ref[...] = jnp.maximum(s, 0.0)


def kernel(X, edge_index, W, b):
    src = edge_index[0].astype(jnp.int32)
    dst = edge_index[1].astype(jnp.int32)
    dst3d = dst.reshape(NW, NCH, KC)

    zerosd = jnp.zeros((RLAST, D), jnp.float32)

    degflat = _deg_kernel(dst3d)
    deg = degflat.reshape(NC, N, 1)

    z = pl.pallas_call(
        _mm_body,
        out_shape=jax.ShapeDtypeStruct((N, D), jnp.float32),
    )(X, W.T, b.reshape(1, D), deg)

    acc = _gather_scatter_kernel(z, src, dst3d, zerosd)

    out = pl.pallas_call(
        _final_body,
        out_shape=jax.ShapeDtypeStruct((N, D), jnp.float32),
    )(acc, z, deg)
    return out
